# Initial kernel scaffold; baseline (speedup 1.0000x reference)
import functools

import jax
import jax.numpy as jnp
from jax import lax
from jax.experimental import pallas as pl
from jax.experimental.pallas import tpu as pltpu
from jax.experimental.pallas import tpu_sc as plsc

_N = 100000
_D = 480
_DS = 288
_NC, _NS = 2, 16
_NW = _NC * _NS
_RPW = _N // _NW
_CHUNK = 25
_NCH = _RPW // _CHUNK
_L = 16


def _combine_body(nf_hbm, dl_hbm, sc_hbm, out_hbm, nf_v, dl_v, sc_v):
    wid = lax.axis_index("s") * _NC + lax.axis_index("c")
    base = wid * _RPW

    def chunk(ci, carry):
        row0 = base + ci * _CHUNK
        pltpu.sync_copy(nf_hbm.at[pl.ds(row0, _CHUNK)], nf_v)
        pltpu.sync_copy(dl_hbm.at[pl.ds(row0, _CHUNK)], dl_v)
        pltpu.sync_copy(sc_hbm.at[pl.ds(row0, _CHUNK)], sc_v)

        def row(r, c2):
            for j in range(_D // _L):
                plsc.addupdate(nf_v.at[r, pl.ds(j * _L, _L)],
                               dl_v[r, pl.ds(j * _L, _L)])
            for j in range(128 // _L):
                plsc.addupdate(nf_v.at[r, pl.ds(j * _L, _L)],
                               sc_v[r, pl.ds(j * _L, _L)])
            for j in range(160 // _L):
                plsc.addupdate(nf_v.at[r, pl.ds(320 + j * _L, _L)],
                               sc_v[r, pl.ds(128 + j * _L, _L)])
            return c2

        lax.fori_loop(0, _CHUNK, row, 0)
        pltpu.sync_copy(nf_v, out_hbm.at[pl.ds(row0, _CHUNK)])
        return carry

    lax.fori_loop(0, _NCH, chunk, 0)


@jax.jit
def kernel(node_features, node_features_delta, node_scalars):
    run = pl.kernel(
        _combine_body,
        out_type=jax.ShapeDtypeStruct((_N, _D), jnp.float32),
        mesh=plsc.VectorSubcoreMesh(core_axis_name="c", subcore_axis_name="s",
                                    num_cores=_NC, num_subcores=_NS),
        scratch_types=[
            pltpu.VMEM((_CHUNK, _D), jnp.float32),
            pltpu.VMEM((_CHUNK, _D), jnp.float32),
            pltpu.VMEM((_CHUNK, _DS), jnp.float32),
        ],
    )
    return run(node_features, node_features_delta, node_scalars)


# speedup vs baseline: 1.9588x; 1.9588x over previous
import functools

import jax
import jax.numpy as jnp
from jax import lax
from jax.experimental import pallas as pl
from jax.experimental.pallas import tpu as pltpu
from jax.experimental.pallas import tpu_sc as plsc

_N = 100000
_D = 480
_DS = 288
_NC, _NS = 2, 16
_NW = _NC * _NS
_CHUNK = 32
_TOTCH = _N // _CHUNK
_L = 16


def _combine_body(nf_hbm, dl_hbm, sc_hbm, out_hbm, nf_v, dl_v, sc_v):
    wid = lax.axis_index("s") * _NC + lax.axis_index("c")
    nch = (_TOTCH - wid + _NW - 1) // _NW

    def chunk(ci, carry):
        row0 = (ci * _NW + wid) * _CHUNK
        pltpu.sync_copy(nf_hbm.at[pl.ds(row0, _CHUNK)], nf_v)
        pltpu.sync_copy(dl_hbm.at[pl.ds(row0, _CHUNK)], dl_v)
        pltpu.sync_copy(sc_hbm.at[pl.ds(row0, _CHUNK)], sc_v)

        def row(r, c2):
            for j in range(_D // _L):
                plsc.addupdate(nf_v.at[r, pl.ds(j * _L, _L)],
                               dl_v[r, pl.ds(j * _L, _L)])
            for j in range(128 // _L):
                plsc.addupdate(nf_v.at[r, pl.ds(j * _L, _L)],
                               sc_v[r, pl.ds(j * _L, _L)])
            for j in range(160 // _L):
                plsc.addupdate(nf_v.at[r, pl.ds(320 + j * _L, _L)],
                               sc_v[r, pl.ds(128 + j * _L, _L)])
            return c2

        lax.fori_loop(0, _CHUNK, row, 0)
        pltpu.sync_copy(nf_v, out_hbm.at[pl.ds(row0, _CHUNK)])
        return carry

    lax.fori_loop(0, nch, chunk, 0)


@jax.jit
def kernel(node_features, node_features_delta, node_scalars):
    run = pl.kernel(
        _combine_body,
        out_type=jax.ShapeDtypeStruct((_N, _D), jnp.float32),
        mesh=plsc.VectorSubcoreMesh(core_axis_name="c", subcore_axis_name="s",
                                    num_cores=_NC, num_subcores=_NS),
        scratch_types=[
            pltpu.VMEM((_CHUNK, _D), jnp.float32),
            pltpu.VMEM((_CHUNK, _D), jnp.float32),
            pltpu.VMEM((_CHUNK, _DS), jnp.float32),
        ],
    )
    return run(node_features, node_features_delta, node_scalars)


# speedup vs baseline: 2.4233x; 1.2371x over previous
import functools

import jax
import jax.numpy as jnp
from jax import lax
from jax.experimental import pallas as pl
from jax.experimental.pallas import tpu as pltpu
from jax.experimental.pallas import tpu_sc as plsc

_N = 100000
_D = 480
_DS = 288
_NC, _NS = 2, 16
_NW = _NC * _NS
_CHUNK = 32
_TOTCH = _N // _CHUNK
_L = 16


def _combine_body(nf_hbm, dl_hbm, sc_hbm, out_hbm, nf_v, dl_v, sc_v):
    wid = lax.axis_index("s") * _NC + lax.axis_index("c")
    nch = (_TOTCH - wid + _NW - 1) // _NW

    def chunk(ci, carry):
        row0 = (ci * _NW + wid) * _CHUNK
        pltpu.sync_copy(nf_hbm.at[pl.ds(row0, _CHUNK)], nf_v)
        pltpu.sync_copy(dl_hbm.at[pl.ds(row0, _CHUNK)], dl_v)
        pltpu.sync_copy(sc_hbm.at[pl.ds(row0, _CHUNK)], sc_v)

        @plsc.parallel_loop(0, _CHUNK, step=1, unroll=4)
        def row(r):
            for j in range(_D // _L):
                plsc.addupdate(nf_v.at[r, pl.ds(j * _L, _L)],
                               dl_v[r, pl.ds(j * _L, _L)])
            for j in range(128 // _L):
                plsc.addupdate(nf_v.at[r, pl.ds(j * _L, _L)],
                               sc_v[r, pl.ds(j * _L, _L)])
            for j in range(160 // _L):
                plsc.addupdate(nf_v.at[r, pl.ds(320 + j * _L, _L)],
                               sc_v[r, pl.ds(128 + j * _L, _L)])
        pltpu.sync_copy(nf_v, out_hbm.at[pl.ds(row0, _CHUNK)])
        return carry

    lax.fori_loop(0, nch, chunk, 0)


@jax.jit
def kernel(node_features, node_features_delta, node_scalars):
    run = pl.kernel(
        _combine_body,
        out_type=jax.ShapeDtypeStruct((_N, _D), jnp.float32),
        mesh=plsc.VectorSubcoreMesh(core_axis_name="c", subcore_axis_name="s",
                                    num_cores=_NC, num_subcores=_NS),
        scratch_types=[
            pltpu.VMEM((_CHUNK, _D), jnp.float32),
            pltpu.VMEM((_CHUNK, _D), jnp.float32),
            pltpu.VMEM((_CHUNK, _DS), jnp.float32),
        ],
    )
    return run(node_features, node_features_delta, node_scalars)
